# baseline (speedup 1.0000x reference)
import functools

import jax
import jax.numpy as jnp
from jax import lax
from jax.experimental import pallas as pl
from jax.experimental.pallas import tpu as pltpu
from jax.experimental.pallas import tpu_sc as plsc

D_MODEL = 64
NUM_CORES = 2
NUM_SUBCORES = 16
LANES = 16
NUM_WORKERS = NUM_CORES * NUM_SUBCORES
CHUNK = 128


def _emb_body(idx_hbm, w_hbm, out_hbm, idx_v, buf, sem, *, n_chunks):
    wid = lax.axis_index("s") * NUM_CORES + lax.axis_index("c")
    chunk0 = wid * n_chunks
    pltpu.sync_copy(idx_hbm.at[pl.ds(chunk0, n_chunks)], idx_v)

    def chunk_body(j, carry):
        pltpu.async_copy(w_hbm.at[idx_v.at[j]], buf, sem).wait()
        for vi in range(CHUNK // LANES):
            v = idx_v[j, pl.ds(vi * LANES, LANES)]
            m = v == 0
            nzero = plsc.all_reduce_population_count(m)

            @pl.when(nzero[0] > 0)
            def _zero_rows(m=m, vi=vi):
                rows = vi * LANES + lax.iota(jnp.int32, LANES)
                zeros = jnp.zeros((LANES,), jnp.float32)

                def col_body(c, cc):
                    cols = jnp.full((LANES,), c, jnp.int32)
                    plsc.store_scatter(buf, [rows, cols], zeros, mask=m)
                    return cc

                lax.fori_loop(0, D_MODEL, col_body, 0)

        pltpu.sync_copy(buf, out_hbm.at[pl.ds((chunk0 + j) * CHUNK, CHUNK)])
        return carry

    lax.fori_loop(0, n_chunks, chunk_body, 0)


def kernel(tokens, weight):
    batch, hist = tokens.shape
    n_rows = batch * hist
    assert n_rows % (NUM_WORKERS * CHUNK) == 0
    n_chunks = n_rows // (NUM_WORKERS * CHUNK)
    idx = tokens.reshape(n_rows // CHUNK, CHUNK).astype(jnp.int32)

    emb = functools.partial(
        pl.kernel,
        out_type=jax.ShapeDtypeStruct((n_rows, D_MODEL), jnp.float32),
        mesh=plsc.VectorSubcoreMesh(core_axis_name="c", subcore_axis_name="s"),
        compiler_params=pltpu.CompilerParams(
            needs_layout_passes=False, use_tc_tiling_on_sc=False
        ),
        scratch_types=[
            pltpu.VMEM((n_chunks, CHUNK), jnp.int32),
            pltpu.VMEM((CHUNK, D_MODEL), jnp.float32),
            pltpu.SemaphoreType.DMA,
        ],
    )(functools.partial(_emb_body, n_chunks=n_chunks))

    out = emb(idx, weight)
    return out.reshape(batch, hist, D_MODEL)


# speedup vs baseline: 1.1420x; 1.1420x over previous
import functools

import jax
import jax.numpy as jnp
from jax import lax
from jax.experimental import pallas as pl
from jax.experimental.pallas import tpu as pltpu
from jax.experimental.pallas import tpu_sc as plsc

D_MODEL = 64
NUM_CORES = 2
NUM_SUBCORES = 16
LANES = 16
NUM_WORKERS = NUM_CORES * NUM_SUBCORES
CHUNK = 128
NBUF = 4
SKEW = 2


def _fixup_padding(idx_v, buf, b, j):
    for vi in range(CHUNK // LANES):
        v = idx_v[j, pl.ds(vi * LANES, LANES)]
        m = v == 0
        nzero = plsc.all_reduce_population_count(m)

        @pl.when(nzero[0] > 0)
        def _zero_rows(m=m, vi=vi):
            rows = vi * LANES + lax.iota(jnp.int32, LANES)
            zeros = jnp.zeros((LANES,), jnp.float32)

            def col_body(c, cc):
                cols = jnp.full((LANES,), c, jnp.int32)
                plsc.store_scatter(buf.at[b], [rows, cols], zeros, mask=m)
                return cc

            lax.fori_loop(0, D_MODEL, col_body, 0)


def _emb_body(idx_hbm, w_hbm, out_hbm, idx_v, buf, gsem, wsem, *, n_chunks):
    wid = lax.axis_index("s") * NUM_CORES + lax.axis_index("c")
    chunk0 = wid * n_chunks
    pltpu.sync_copy(idx_hbm.at[pl.ds(chunk0, n_chunks)], idx_v)

    def pipe_body(i, carry):
        j_w = i - NBUF

        @pl.when((j_w >= 0) & (j_w < n_chunks))
        def _wait_wb():
            pltpu.make_async_copy(
                buf.at[j_w % NBUF],
                out_hbm.at[pl.ds((chunk0 + j_w) * CHUNK, CHUNK)],
                wsem.at[j_w % NBUF],
            ).wait()

        @pl.when(i < n_chunks)
        def _start_gather():
            pltpu.async_copy(
                w_hbm.at[idx_v.at[i]], buf.at[i % NBUF], gsem.at[i % NBUF]
            )

        j = i - SKEW

        @pl.when((j >= 0) & (j < n_chunks))
        def _consume():
            b = j % NBUF
            pltpu.make_async_copy(
                w_hbm.at[idx_v.at[j]], buf.at[b], gsem.at[b]
            ).wait()
            _fixup_padding(idx_v, buf, b, j)
            pltpu.async_copy(
                buf.at[b],
                out_hbm.at[pl.ds((chunk0 + j) * CHUNK, CHUNK)],
                wsem.at[b],
            )

        return carry

    lax.fori_loop(0, n_chunks + NBUF, pipe_body, 0)


def kernel(tokens, weight):
    batch, hist = tokens.shape
    n_rows = batch * hist
    assert n_rows % (NUM_WORKERS * CHUNK) == 0
    n_chunks = n_rows // (NUM_WORKERS * CHUNK)
    idx = tokens.reshape(n_rows // CHUNK, CHUNK).astype(jnp.int32)

    emb = functools.partial(
        pl.kernel,
        out_type=jax.ShapeDtypeStruct((n_rows, D_MODEL), jnp.float32),
        mesh=plsc.VectorSubcoreMesh(core_axis_name="c", subcore_axis_name="s"),
        compiler_params=pltpu.CompilerParams(
            needs_layout_passes=False, use_tc_tiling_on_sc=False
        ),
        scratch_types=[
            pltpu.VMEM((n_chunks, CHUNK), jnp.int32),
            pltpu.VMEM((NBUF, CHUNK, D_MODEL), jnp.float32),
            pltpu.SemaphoreType.DMA((NBUF,)),
            pltpu.SemaphoreType.DMA((NBUF,)),
        ],
    )(functools.partial(_emb_body, n_chunks=n_chunks))

    out = emb(idx, weight)
    return out.reshape(batch, hist, D_MODEL)
